# Initial kernel scaffold; baseline (speedup 1.0000x reference)
import jax
import jax.numpy as jnp
from jax.experimental import pallas as pl


def kernel(x, edge_attr, edge_index, batch, params):
    raise NotImplementedError("write your pallas kernel here")


# speedup vs baseline: 1.4045x; 1.4045x over previous
import functools
import math

import jax
import jax.numpy as jnp
from jax import lax
from jax.experimental import pallas as pl
from jax.experimental.pallas import tpu as pltpu

N = 10000
E = 320000
D = 128
G = 64
ATOM_K = 256
BOND_K = 128

NODE_BLK = 1000
EDGE_BLK = 2000


def _gelu(x):
    return 0.5 * x * (1.0 + lax.erf(x * (1.0 / math.sqrt(2.0))))


def _ln_rows(x, g, b):
    mu = jnp.mean(x, axis=-1, keepdims=True)
    var = jnp.mean((x - mu) ** 2, axis=-1, keepdims=True)
    return (x - mu) * lax.rsqrt(var + 1e-5) * g + b


def _encoder_body(idx_ref, offs_ref, emb_ref, w1_ref, b1_ref, w2_ref, b2_ref,
                  lng_ref, lnb_ref, out_ref, *, kdim):
    idx = idx_ref[...]
    offs = offs_ref[0, :]
    B = idx.shape[0]
    iota = lax.broadcasted_iota(jnp.int32, (B, kdim), 1)
    oh = jnp.zeros((B, kdim), jnp.float32)
    for f in range(idx.shape[1]):
        col = (idx[:, f] + offs[f])[:, None]
        oh = oh + (iota == col).astype(jnp.float32)
    h = jnp.dot(oh, emb_ref[...], preferred_element_type=jnp.float32)
    h = _ln_rows(h, lng_ref[0, :], lnb_ref[0, :])
    t = jnp.dot(h, w1_ref[...], preferred_element_type=jnp.float32) + b1_ref[0, :]
    t = _gelu(t)
    out_ref[...] = (jnp.dot(t, w2_ref[...], preferred_element_type=jnp.float32)
                    + b2_ref[0, :])


def _encode_tc(idx, p, kdim, blk):
    M, F = idx.shape
    vocabs = [t.shape[0] for t in p["embs"]]
    offs = [0]
    for v in vocabs[:-1]:
        offs.append(offs[-1] + v)
    emb = jnp.concatenate(p["embs"], axis=0)
    emb = jnp.pad(emb, ((0, kdim - emb.shape[0]), (0, 0)))
    offs = jnp.array(offs, jnp.int32)[None, :]
    grid = M // blk
    return pl.pallas_call(
        functools.partial(_encoder_body, kdim=kdim),
        grid=(grid,),
        in_specs=[
            pl.BlockSpec((blk, F), lambda i: (i, 0)),
            pl.BlockSpec((1, F), lambda i: (0, 0)),
            pl.BlockSpec((kdim, D), lambda i: (0, 0)),
            pl.BlockSpec((D, D), lambda i: (0, 0)),
            pl.BlockSpec((1, D), lambda i: (0, 0)),
            pl.BlockSpec((D, D), lambda i: (0, 0)),
            pl.BlockSpec((1, D), lambda i: (0, 0)),
            pl.BlockSpec((1, D), lambda i: (0, 0)),
            pl.BlockSpec((1, D), lambda i: (0, 0)),
        ],
        out_specs=pl.BlockSpec((blk, D), lambda i: (i, 0)),
        out_shape=jax.ShapeDtypeStruct((M, D), jnp.float32),
    )(idx, offs, emb, p["w1"], p["b1"][None, :], p["w2"], p["b2"][None, :],
      p["ln_g"][None, :], p["ln_b"][None, :])


def _node_body(h_ref, a0_ref, a1_ref, w1_ref, b1_ref, w2_ref, b2_ref,
               lng_ref, lnb_ref, out_ref):
    h = h_ref[...]
    z = h + a0_ref[0] + a1_ref[0]
    t = jnp.maximum(
        jnp.dot(z, w1_ref[...], preferred_element_type=jnp.float32)
        + b1_ref[0, :], 0.0)
    t = jnp.dot(t, w2_ref[...], preferred_element_type=jnp.float32) + b2_ref[0, :]
    t = _gelu(t)
    out_ref[...] = _ln_rows(t + h, lng_ref[0, :], lnb_ref[0, :])


def _node_update_tc(h, agg2, lp):
    grid = N // NODE_BLK
    return pl.pallas_call(
        _node_body,
        grid=(grid,),
        in_specs=[
            pl.BlockSpec((NODE_BLK, D), lambda i: (i, 0)),
            pl.BlockSpec((1, NODE_BLK, D), lambda i: (0, i, 0)),
            pl.BlockSpec((1, NODE_BLK, D), lambda i: (1, i, 0)),
            pl.BlockSpec((D, D), lambda i: (0, 0)),
            pl.BlockSpec((1, D), lambda i: (0, 0)),
            pl.BlockSpec((D, D), lambda i: (0, 0)),
            pl.BlockSpec((1, D), lambda i: (0, 0)),
            pl.BlockSpec((1, D), lambda i: (0, 0)),
            pl.BlockSpec((1, D), lambda i: (0, 0)),
        ],
        out_specs=pl.BlockSpec((NODE_BLK, D), lambda i: (i, 0)),
        out_shape=jax.ShapeDtypeStruct((N, D), jnp.float32),
    )(h, agg2, agg2, lp["w1"], lp["b1"][None, :], lp["w2"], lp["b2"][None, :],
      lp["ln_g"][None, :], lp["ln_b"][None, :])


def _pool_body(batch_ref, h_ref, pw_ref, pb_ref, out_ref, sums_ref, cnts_ref):
    i = pl.program_id(0)

    @pl.when(i == 0)
    def _init():
        sums_ref[...] = jnp.zeros_like(sums_ref)
        cnts_ref[...] = jnp.zeros_like(cnts_ref)

    brow = batch_ref[0, :, :]
    gids = lax.broadcasted_iota(jnp.int32, (G, brow.shape[1]), 0)
    oh = (brow == gids).astype(jnp.float32)
    sums_ref[...] += jnp.dot(oh, h_ref[...], preferred_element_type=jnp.float32)
    cnts_ref[...] += jnp.broadcast_to(
        jnp.sum(oh, axis=1, keepdims=True), cnts_ref.shape)

    @pl.when(i == pl.num_programs(0) - 1)
    def _final():
        g = sums_ref[...] / jnp.maximum(cnts_ref[...], 1.0)
        g = jnp.dot(g, pw_ref[...], preferred_element_type=jnp.float32) + pb_ref[0, :]
        nrm = jnp.sqrt(jnp.sum(g * g, axis=-1, keepdims=True))
        out_ref[...] = g / jnp.maximum(nrm, 1e-12)


def _pool_tc(h, batch, pw, pb):
    grid = N // NODE_BLK
    batch3 = batch.astype(jnp.int32).reshape(grid, 1, NODE_BLK)
    return pl.pallas_call(
        _pool_body,
        grid=(grid,),
        in_specs=[
            pl.BlockSpec((1, 1, NODE_BLK), lambda i: (i, 0, 0)),
            pl.BlockSpec((NODE_BLK, D), lambda i: (i, 0)),
            pl.BlockSpec((D, D), lambda i: (0, 0)),
            pl.BlockSpec((1, D), lambda i: (0, 0)),
        ],
        out_specs=pl.BlockSpec((G, D), lambda i: (0, 0)),
        out_shape=jax.ShapeDtypeStruct((G, D), jnp.float32),
        scratch_shapes=[pltpu.VMEM((G, D), jnp.float32),
                        pltpu.VMEM((G, D), jnp.float32)],
    )(batch3, h, pw, pb[None, :])


def _edge_stage(h, e, src, dst):
    msg = jax.nn.relu(h[src] + e)
    agg = jnp.zeros_like(h).at[dst].add(msg)
    return jnp.stack([agg, jnp.zeros_like(agg)])


def kernel(x, edge_attr, edge_index, batch, params):
    x = x.astype(jnp.int32)
    edge_attr = edge_attr.astype(jnp.int32)
    src = edge_index[0].astype(jnp.int32)
    dst = edge_index[1].astype(jnp.int32)

    h = _encode_tc(x, params["atom"], ATOM_K, NODE_BLK)
    e = _encode_tc(edge_attr, params["bond"], BOND_K, EDGE_BLK)

    for lp in params["layers"]:
        agg2 = _edge_stage(h, e, src, dst)
        h = _node_update_tc(h, agg2, lp)

    return _pool_tc(h, batch, params["proj_w"], params["proj_b"])


# speedup vs baseline: 4.1030x; 2.9213x over previous
import functools
import math

import jax
import jax.numpy as jnp
from jax import lax
from jax.experimental import pallas as pl
from jax.experimental.pallas import tpu as pltpu
from jax.experimental.pallas import tpu_sc as plsc

N = 10000
E = 320000
D = 128
G = 64
ATOM_K = 256
BOND_K = 128

NODE_BLK = 1000
EDGE_BLK = 2000


def _gelu(x):
    return 0.5 * x * (1.0 + lax.erf(x * (1.0 / math.sqrt(2.0))))


def _ln_rows(x, g, b):
    mu = jnp.mean(x, axis=-1, keepdims=True)
    var = jnp.mean((x - mu) ** 2, axis=-1, keepdims=True)
    return (x - mu) * lax.rsqrt(var + 1e-5) * g + b


def _encoder_body(idx_ref, offs_ref, emb_ref, w1_ref, b1_ref, w2_ref, b2_ref,
                  lng_ref, lnb_ref, out_ref, *, kdim):
    idx = idx_ref[...]
    offs = offs_ref[0, :]
    B = idx.shape[0]
    iota = lax.broadcasted_iota(jnp.int32, (B, kdim), 1)
    oh = jnp.zeros((B, kdim), jnp.float32)
    for f in range(idx.shape[1]):
        col = (idx[:, f] + offs[f])[:, None]
        oh = oh + (iota == col).astype(jnp.float32)
    h = jnp.dot(oh, emb_ref[...], preferred_element_type=jnp.float32)
    h = _ln_rows(h, lng_ref[0, :], lnb_ref[0, :])
    t = jnp.dot(h, w1_ref[...], preferred_element_type=jnp.float32) + b1_ref[0, :]
    t = _gelu(t)
    out_ref[...] = (jnp.dot(t, w2_ref[...], preferred_element_type=jnp.float32)
                    + b2_ref[0, :])


def _encode_tc(idx, p, kdim, blk):
    M, F = idx.shape
    vocabs = [t.shape[0] for t in p["embs"]]
    offs = [0]
    for v in vocabs[:-1]:
        offs.append(offs[-1] + v)
    emb = jnp.concatenate(p["embs"], axis=0)
    emb = jnp.pad(emb, ((0, kdim - emb.shape[0]), (0, 0)))
    offs = jnp.array(offs, jnp.int32)[None, :]
    grid = M // blk
    return pl.pallas_call(
        functools.partial(_encoder_body, kdim=kdim),
        grid=(grid,),
        in_specs=[
            pl.BlockSpec((blk, F), lambda i: (i, 0)),
            pl.BlockSpec((1, F), lambda i: (0, 0)),
            pl.BlockSpec((kdim, D), lambda i: (0, 0)),
            pl.BlockSpec((D, D), lambda i: (0, 0)),
            pl.BlockSpec((1, D), lambda i: (0, 0)),
            pl.BlockSpec((D, D), lambda i: (0, 0)),
            pl.BlockSpec((1, D), lambda i: (0, 0)),
            pl.BlockSpec((1, D), lambda i: (0, 0)),
            pl.BlockSpec((1, D), lambda i: (0, 0)),
        ],
        out_specs=pl.BlockSpec((blk, D), lambda i: (i, 0)),
        out_shape=jax.ShapeDtypeStruct((M, D), jnp.float32),
    )(idx, offs, emb, p["w1"], p["b1"][None, :], p["w2"], p["b2"][None, :],
      p["ln_g"][None, :], p["ln_b"][None, :])


def _node_body(h_ref, a0_ref, a1_ref, w1_ref, b1_ref, w2_ref, b2_ref,
               lng_ref, lnb_ref, out_ref):
    h = h_ref[...]
    z = h + a0_ref[0] + a1_ref[0]
    t = jnp.maximum(
        jnp.dot(z, w1_ref[...], preferred_element_type=jnp.float32)
        + b1_ref[0, :], 0.0)
    t = jnp.dot(t, w2_ref[...], preferred_element_type=jnp.float32) + b2_ref[0, :]
    t = _gelu(t)
    out_ref[...] = _ln_rows(t + h, lng_ref[0, :], lnb_ref[0, :])


def _node_update_tc(h, agg2, lp):
    grid = N // NODE_BLK
    return pl.pallas_call(
        _node_body,
        grid=(grid,),
        in_specs=[
            pl.BlockSpec((NODE_BLK, D), lambda i: (i, 0)),
            pl.BlockSpec((1, NODE_BLK, D), lambda i: (0, i, 0)),
            pl.BlockSpec((1, NODE_BLK, D), lambda i: (1, i, 0)),
            pl.BlockSpec((D, D), lambda i: (0, 0)),
            pl.BlockSpec((1, D), lambda i: (0, 0)),
            pl.BlockSpec((D, D), lambda i: (0, 0)),
            pl.BlockSpec((1, D), lambda i: (0, 0)),
            pl.BlockSpec((1, D), lambda i: (0, 0)),
            pl.BlockSpec((1, D), lambda i: (0, 0)),
        ],
        out_specs=pl.BlockSpec((NODE_BLK, D), lambda i: (i, 0)),
        out_shape=jax.ShapeDtypeStruct((N, D), jnp.float32),
    )(h, agg2, agg2, lp["w1"], lp["b1"][None, :], lp["w2"], lp["b2"][None, :],
      lp["ln_g"][None, :], lp["ln_b"][None, :])


def _pool_body(batch_ref, h_ref, pw_ref, pb_ref, out_ref, sums_ref, cnts_ref):
    i = pl.program_id(0)

    @pl.when(i == 0)
    def _init():
        sums_ref[...] = jnp.zeros_like(sums_ref)
        cnts_ref[...] = jnp.zeros_like(cnts_ref)

    brow = batch_ref[0, :, :]
    gids = lax.broadcasted_iota(jnp.int32, (G, brow.shape[1]), 0)
    oh = (brow == gids).astype(jnp.float32)
    sums_ref[...] += jnp.dot(oh, h_ref[...], preferred_element_type=jnp.float32)
    cnts_ref[...] += jnp.broadcast_to(
        jnp.sum(oh, axis=1, keepdims=True), cnts_ref.shape)

    @pl.when(i == pl.num_programs(0) - 1)
    def _final():
        g = sums_ref[...] / jnp.maximum(cnts_ref[...], 1.0)
        g = jnp.dot(g, pw_ref[...], preferred_element_type=jnp.float32) + pb_ref[0, :]
        nrm = jnp.sqrt(jnp.sum(g * g, axis=-1, keepdims=True))
        out_ref[...] = g / jnp.maximum(nrm, 1e-12)


def _pool_tc(h, batch, pw, pb):
    grid = N // NODE_BLK
    batch3 = batch.astype(jnp.int32).reshape(grid, 1, NODE_BLK)
    return pl.pallas_call(
        _pool_body,
        grid=(grid,),
        in_specs=[
            pl.BlockSpec((1, 1, NODE_BLK), lambda i: (i, 0, 0)),
            pl.BlockSpec((NODE_BLK, D), lambda i: (i, 0)),
            pl.BlockSpec((D, D), lambda i: (0, 0)),
            pl.BlockSpec((1, D), lambda i: (0, 0)),
        ],
        out_specs=pl.BlockSpec((G, D), lambda i: (0, 0)),
        out_shape=jax.ShapeDtypeStruct((G, D), jnp.float32),
        scratch_shapes=[pltpu.VMEM((G, D), jnp.float32),
                        pltpu.VMEM((G, D), jnp.float32)],
    )(batch3, h, pw, pb[None, :])


NC = 2
NS = 16
NW = NC * NS
EPW = E // NW
CHUNK = 80
NCHUNKS = EPW // CHUNK
N_PAD = 10240
ROWS_PER_SID = N_PAD // NS
STAGE_ROWS = 128


def _edge_body(h_hbm, src_hbm, dst_hbm, e_hbm, out_hbm,
               agg_sh, src_v, dst_v, rows_v, e_v, stage_v, gsem):
    cid = lax.axis_index("c")
    sid = lax.axis_index("s")
    wid = sid * NC + cid

    def _zrow(r, _):
        for j in range(8):
            stage_v[r, pl.ds(j * 16, 16)] = jnp.zeros((16,), jnp.float32)
        return 0
    lax.fori_loop(0, STAGE_ROWS, _zrow, 0)
    row0 = sid * ROWS_PER_SID
    for i in range(ROWS_PER_SID // STAGE_ROWS):
        pltpu.sync_copy(stage_v,
                        agg_sh.at[pl.ds(row0 + i * STAGE_ROWS, STAGE_ROWS), :])
    plsc.subcore_barrier()

    ebase = wid * EPW

    def _chunk(k, _):
        base = ebase + k * CHUNK
        pltpu.sync_copy(src_hbm.at[pl.ds(base, CHUNK)], src_v)
        pltpu.sync_copy(dst_hbm.at[pl.ds(base, CHUNK)], dst_v)
        gd = pltpu.async_copy(h_hbm.at[src_v], rows_v, gsem)
        pltpu.sync_copy(e_hbm.at[pl.ds(base, CHUNK), :], e_v)
        gd.wait()

        def _row(r, _):
            for j in range(8):
                s = pl.ds(j * 16, 16)
                rows_v[r, s] = jnp.maximum(rows_v[r, s] + e_v[r, s], 0.0)
            return 0
        lax.fori_loop(0, CHUNK, _row, 0)
        pltpu.sync_copy(rows_v, agg_sh.at[dst_v], add=True)
        return 0
    lax.fori_loop(0, NCHUNKS, _chunk, 0)
    plsc.subcore_barrier()

    for i in range(ROWS_PER_SID // STAGE_ROWS):
        r0 = row0 + i * STAGE_ROWS
        pltpu.sync_copy(agg_sh.at[pl.ds(r0, STAGE_ROWS), :], stage_v)
        pltpu.sync_copy(stage_v, out_hbm.at[cid, pl.ds(r0, STAGE_ROWS), :])


_edge_sc = functools.partial(
    pl.kernel,
    out_type=jax.ShapeDtypeStruct((NC, N_PAD, D), jnp.float32),
    mesh=plsc.VectorSubcoreMesh(core_axis_name="c", subcore_axis_name="s"),
    scratch_types=[
        pltpu.VMEM_SHARED((N_PAD, D), jnp.float32),
        pltpu.VMEM((CHUNK,), jnp.int32),
        pltpu.VMEM((CHUNK,), jnp.int32),
        pltpu.VMEM((CHUNK, D), jnp.float32),
        pltpu.VMEM((CHUNK, D), jnp.float32),
        pltpu.VMEM((STAGE_ROWS, D), jnp.float32),
        pltpu.SemaphoreType.DMA,
    ],
)(_edge_body)


def _edge_stage(h, e, src, dst):
    return _edge_sc(h, src, dst, e)


def kernel(x, edge_attr, edge_index, batch, params):
    x = x.astype(jnp.int32)
    edge_attr = edge_attr.astype(jnp.int32)
    src = edge_index[0].astype(jnp.int32)
    dst = edge_index[1].astype(jnp.int32)

    h = _encode_tc(x, params["atom"], ATOM_K, NODE_BLK)
    e = _encode_tc(edge_attr, params["bond"], BOND_K, EDGE_BLK)

    for lp in params["layers"]:
        agg2 = _edge_stage(h, e, src, dst)
        h = _node_update_tc(h, agg2, lp)

    return _pool_tc(h, batch, params["proj_w"], params["proj_b"])


# speedup vs baseline: 6.8994x; 1.6815x over previous
import functools
import math

import jax
import jax.numpy as jnp
from jax import lax
from jax.experimental import pallas as pl
from jax.experimental.pallas import tpu as pltpu
from jax.experimental.pallas import tpu_sc as plsc

N = 10000
E = 320000
D = 128
G = 64
ATOM_K = 256
BOND_K = 128

NODE_BLK = 1000
EDGE_BLK = 2000


def _gelu(x):
    return 0.5 * x * (1.0 + lax.erf(x * (1.0 / math.sqrt(2.0))))


def _ln_rows(x, g, b):
    mu = jnp.mean(x, axis=-1, keepdims=True)
    var = jnp.mean((x - mu) ** 2, axis=-1, keepdims=True)
    return (x - mu) * lax.rsqrt(var + 1e-5) * g + b


def _encoder_body(idx_ref, offs_ref, emb_ref, w1_ref, b1_ref, w2_ref, b2_ref,
                  lng_ref, lnb_ref, out_ref, *, kdim):
    idx = idx_ref[...]
    offs = offs_ref[0, :]
    B = idx.shape[0]
    iota = lax.broadcasted_iota(jnp.int32, (B, kdim), 1)
    oh = jnp.zeros((B, kdim), jnp.float32)
    for f in range(idx.shape[1]):
        col = (idx[:, f] + offs[f])[:, None]
        oh = oh + (iota == col).astype(jnp.float32)
    h = jnp.dot(oh, emb_ref[...], preferred_element_type=jnp.float32)
    h = _ln_rows(h, lng_ref[0, :], lnb_ref[0, :])
    t = jnp.dot(h, w1_ref[...], preferred_element_type=jnp.float32) + b1_ref[0, :]
    t = _gelu(t)
    out_ref[...] = (jnp.dot(t, w2_ref[...], preferred_element_type=jnp.float32)
                    + b2_ref[0, :])


def _encode_tc(idx, p, kdim, blk):
    M, F = idx.shape
    vocabs = [t.shape[0] for t in p["embs"]]
    offs = [0]
    for v in vocabs[:-1]:
        offs.append(offs[-1] + v)
    emb = jnp.concatenate(p["embs"], axis=0)
    emb = jnp.pad(emb, ((0, kdim - emb.shape[0]), (0, 0)))
    offs = jnp.array(offs, jnp.int32)[None, :]
    grid = M // blk
    return pl.pallas_call(
        functools.partial(_encoder_body, kdim=kdim),
        grid=(grid,),
        in_specs=[
            pl.BlockSpec((blk, F), lambda i: (i, 0)),
            pl.BlockSpec((1, F), lambda i: (0, 0)),
            pl.BlockSpec((kdim, D), lambda i: (0, 0)),
            pl.BlockSpec((D, D), lambda i: (0, 0)),
            pl.BlockSpec((1, D), lambda i: (0, 0)),
            pl.BlockSpec((D, D), lambda i: (0, 0)),
            pl.BlockSpec((1, D), lambda i: (0, 0)),
            pl.BlockSpec((1, D), lambda i: (0, 0)),
            pl.BlockSpec((1, D), lambda i: (0, 0)),
        ],
        out_specs=pl.BlockSpec((blk, D), lambda i: (i, 0)),
        out_shape=jax.ShapeDtypeStruct((M, D), jnp.float32),
    )(idx, offs, emb, p["w1"], p["b1"][None, :], p["w2"], p["b2"][None, :],
      p["ln_g"][None, :], p["ln_b"][None, :])


def _node_body(h_ref, a0_ref, a1_ref, w1_ref, b1_ref, w2_ref, b2_ref,
               lng_ref, lnb_ref, out_ref):
    h = h_ref[...]
    z = h + a0_ref[0] + a1_ref[0]
    t = jnp.maximum(
        jnp.dot(z, w1_ref[...], preferred_element_type=jnp.float32)
        + b1_ref[0, :], 0.0)
    t = jnp.dot(t, w2_ref[...], preferred_element_type=jnp.float32) + b2_ref[0, :]
    t = _gelu(t)
    out_ref[...] = _ln_rows(t + h, lng_ref[0, :], lnb_ref[0, :])


def _node_update_tc(h, agg2, lp):
    grid = N // NODE_BLK
    return pl.pallas_call(
        _node_body,
        grid=(grid,),
        in_specs=[
            pl.BlockSpec((NODE_BLK, D), lambda i: (i, 0)),
            pl.BlockSpec((1, NODE_BLK, D), lambda i: (0, i, 0)),
            pl.BlockSpec((1, NODE_BLK, D), lambda i: (1, i, 0)),
            pl.BlockSpec((D, D), lambda i: (0, 0)),
            pl.BlockSpec((1, D), lambda i: (0, 0)),
            pl.BlockSpec((D, D), lambda i: (0, 0)),
            pl.BlockSpec((1, D), lambda i: (0, 0)),
            pl.BlockSpec((1, D), lambda i: (0, 0)),
            pl.BlockSpec((1, D), lambda i: (0, 0)),
        ],
        out_specs=pl.BlockSpec((NODE_BLK, D), lambda i: (i, 0)),
        out_shape=jax.ShapeDtypeStruct((N, D), jnp.float32),
    )(h, agg2, agg2, lp["w1"], lp["b1"][None, :], lp["w2"], lp["b2"][None, :],
      lp["ln_g"][None, :], lp["ln_b"][None, :])


def _pool_body(batch_ref, h_ref, pw_ref, pb_ref, out_ref, sums_ref, cnts_ref):
    i = pl.program_id(0)

    @pl.when(i == 0)
    def _init():
        sums_ref[...] = jnp.zeros_like(sums_ref)
        cnts_ref[...] = jnp.zeros_like(cnts_ref)

    brow = batch_ref[0, :, :]
    gids = lax.broadcasted_iota(jnp.int32, (G, brow.shape[1]), 0)
    oh = (brow == gids).astype(jnp.float32)
    sums_ref[...] += jnp.dot(oh, h_ref[...], preferred_element_type=jnp.float32)
    cnts_ref[...] += jnp.broadcast_to(
        jnp.sum(oh, axis=1, keepdims=True), cnts_ref.shape)

    @pl.when(i == pl.num_programs(0) - 1)
    def _final():
        g = sums_ref[...] / jnp.maximum(cnts_ref[...], 1.0)
        g = jnp.dot(g, pw_ref[...], preferred_element_type=jnp.float32) + pb_ref[0, :]
        nrm = jnp.sqrt(jnp.sum(g * g, axis=-1, keepdims=True))
        out_ref[...] = g / jnp.maximum(nrm, 1e-12)


def _pool_tc(h, batch, pw, pb):
    grid = N // NODE_BLK
    batch3 = batch.astype(jnp.int32).reshape(grid, 1, NODE_BLK)
    return pl.pallas_call(
        _pool_body,
        grid=(grid,),
        in_specs=[
            pl.BlockSpec((1, 1, NODE_BLK), lambda i: (i, 0, 0)),
            pl.BlockSpec((NODE_BLK, D), lambda i: (i, 0)),
            pl.BlockSpec((D, D), lambda i: (0, 0)),
            pl.BlockSpec((1, D), lambda i: (0, 0)),
        ],
        out_specs=pl.BlockSpec((G, D), lambda i: (0, 0)),
        out_shape=jax.ShapeDtypeStruct((G, D), jnp.float32),
        scratch_shapes=[pltpu.VMEM((G, D), jnp.float32),
                        pltpu.VMEM((G, D), jnp.float32)],
    )(batch3, h, pw, pb[None, :])


NC = 2
NS = 16
NW = NC * NS
EPW = E // NW
CHUNK = 80
NCHUNKS = EPW // CHUNK
N_PAD = 10240
ROWS_PER_SID = N_PAD // NS
STAGE_ROWS = 128


def _edge_body(h_hbm, src_hbm, dst_hbm, e_hbm, out_hbm,
               agg_sh, srcv0, srcv1, srcv2, srcv3,
               dstv0, dstv1, dstv2, dstv3, rows0, rows1,
               ebuf0, ebuf1,
               gsem0, gsem1, esem0, esem1, ssem0, ssem1,
               isem0, isem1, isem2, isem3, dsem0, dsem1, dsem2, dsem3):
    cid = lax.axis_index("c")
    sid = lax.axis_index("s")
    wid = sid * NC + cid
    srcv = (srcv0, srcv1, srcv2, srcv3)
    dstv = (dstv0, dstv1, dstv2, dstv3)
    rows = (rows0, rows1)
    ebuf = (ebuf0, ebuf1)
    gsem = (gsem0, gsem1)
    esem = (esem0, esem1)
    ssem = (ssem0, ssem1)
    isem = (isem0, isem1, isem2, isem3)
    dsem = (dsem0, dsem1, dsem2, dsem3)
    ebase = wid * EPW

    def _zrow(r, _):
        for j in range(8):
            rows0[r, pl.ds(j * 16, 16)] = jnp.zeros((16,), jnp.float32)
        return 0
    lax.fori_loop(0, CHUNK, _zrow, 0)
    row0 = sid * ROWS_PER_SID
    for i in range(ROWS_PER_SID // CHUNK):
        pltpu.sync_copy(rows0, agg_sh.at[pl.ds(row0 + i * CHUNK, CHUNK), :])
    plsc.subcore_barrier()

    def _issue_idx(j, q):
        base = ebase + j * CHUNK
        pltpu.async_copy(src_hbm.at[pl.ds(base, CHUNK)], srcv[q], isem[q])
        pltpu.async_copy(dst_hbm.at[pl.ds(base, CHUNK)], dstv[q], dsem[q])

    def _wait_idx(j, q):
        base = ebase + j * CHUNK
        pltpu.make_async_copy(src_hbm.at[pl.ds(base, CHUNK)], srcv[q],
                              isem[q]).wait()
        pltpu.make_async_copy(dst_hbm.at[pl.ds(base, CHUNK)], dstv[q],
                              dsem[q]).wait()

    def _issue_in(j, q, b):
        pltpu.async_copy(h_hbm.at[srcv[q]], rows[b], gsem[b])
        pltpu.async_copy(e_hbm.at[pl.ds(ebase + j * CHUNK, CHUNK), :],
                         ebuf[b], esem[b])

    def _wait_in(j, q, b):
        pltpu.make_async_copy(h_hbm.at[srcv[q]], rows[b], gsem[b]).wait()
        pltpu.make_async_copy(e_hbm.at[pl.ds(ebase + j * CHUNK, CHUNK), :],
                              ebuf[b], esem[b]).wait()

    def _scatter_wait(q, b):
        pltpu.make_async_copy(rows[b], agg_sh.at[dstv[q]], ssem[b]).wait()

    _issue_idx(0, 0)
    _issue_idx(1, 1)
    _wait_idx(0, 0)
    _issue_in(0, 0, 0)

    def _quad(i, _):
        for b4 in range(4):
            j = 4 * i + b4
            q = b4
            b = b4 % 2

            def _do():
                _wait_in(j, q, b)

                @pl.when(j >= 1)
                def _():
                    _scatter_wait((q - 1) % 4, b ^ 1)

                @pl.when(j + 1 < NCHUNKS)
                def _():
                    _wait_idx(j + 1, (q + 1) % 4)
                    _issue_in(j + 1, (q + 1) % 4, b ^ 1)

                @pl.when(j + 2 < NCHUNKS)
                def _():
                    _issue_idx(j + 2, (q + 2) % 4)

                def _row(r, _):
                    for jj in range(8):
                        s = pl.ds(jj * 16, 16)
                        rows[b][r, s] = jnp.maximum(
                            rows[b][r, s] + ebuf[b][r, s], 0.0)
                    return 0
                lax.fori_loop(0, CHUNK, _row, 0)
                pltpu.async_copy(rows[b], agg_sh.at[dstv[q]], ssem[b],
                                 add=True)

            if b4 == 0:
                _do()
            else:
                pl.when(j < NCHUNKS)(_do)
        return 0
    lax.fori_loop(0, (NCHUNKS + 3) // 4, _quad, 0)

    _scatter_wait((NCHUNKS - 1) % 4, (NCHUNKS - 1) % 2)
    plsc.subcore_barrier()

    for i in range(ROWS_PER_SID // CHUNK):
        r0 = row0 + i * CHUNK
        pltpu.sync_copy(agg_sh.at[pl.ds(r0, CHUNK), :], rows0)
        pltpu.sync_copy(rows0, out_hbm.at[cid, pl.ds(r0, CHUNK), :])


_edge_sc = functools.partial(
    pl.kernel,
    out_type=jax.ShapeDtypeStruct((NC, N_PAD, D), jnp.float32),
    mesh=plsc.VectorSubcoreMesh(core_axis_name="c", subcore_axis_name="s"),
    scratch_types=[pltpu.VMEM_SHARED((N_PAD, D), jnp.float32)]
    + [pltpu.VMEM((CHUNK,), jnp.int32)] * 8
    + [pltpu.VMEM((CHUNK, D), jnp.float32)] * 4
    + [pltpu.SemaphoreType.DMA] * 14,
)(_edge_body)


def _edge_stage(h, e, src, dst):
    return _edge_sc(h, src, dst, e)


def kernel(x, edge_attr, edge_index, batch, params):
    x = x.astype(jnp.int32)
    edge_attr = edge_attr.astype(jnp.int32)
    src = edge_index[0].astype(jnp.int32)
    dst = edge_index[1].astype(jnp.int32)

    h = _encode_tc(x, params["atom"], ATOM_K, NODE_BLK)
    e = _encode_tc(edge_attr, params["bond"], BOND_K, EDGE_BLK)

    for lp in params["layers"]:
        agg2 = _edge_stage(h, e, src, dst)
        h = _node_update_tc(h, agg2, lp)

    return _pool_tc(h, batch, params["proj_w"], params["proj_b"])
